# Initial kernel scaffold; baseline (speedup 1.0000x reference)
import jax
import jax.numpy as jnp
from jax.experimental import pallas as pl


def kernel(inputs, w_gate):
    raise NotImplementedError("write your pallas kernel here")


# speedup vs baseline: 3.8823x; 3.8823x over previous
import functools

import jax
import jax.numpy as jnp
from jax import lax
from jax.experimental import pallas as pl
from jax.experimental.pallas import tpu as pltpu
from jax.experimental.pallas import tpu_sc as plsc

N_TOKENS = 16384
D_MODEL = 4096
NUM_EXPERTS = 64
TOP_K = 8

_NC = 2
_NS = 16
_NW = _NC * _NS
_L = 16

_ROWS_PER_W = N_TOKENS // _NW


def _mm_body(x_ref, w_ref, o_ref):
    o_ref[...] = jnp.dot(x_ref[...], w_ref[...],
                         preferred_element_type=jnp.float32)


def _matmul(inputs, w_gate, bm=256):
    n, d = inputs.shape
    e = w_gate.shape[1]
    return pl.pallas_call(
        _mm_body,
        grid=(n // bm,),
        in_specs=[
            pl.BlockSpec((bm, d), lambda i: (i, 0)),
            pl.BlockSpec((d, e), lambda i: (0, 0)),
        ],
        out_specs=pl.BlockSpec((bm, e), lambda i: (i, 0)),
        out_shape=jax.ShapeDtypeStruct((n, e), jnp.float32),
    )(inputs, w_gate)


def _route_body(rw_hbm, out_hbm, rw_v, out_v):
    wid = lax.axis_index("s") * _NC + lax.axis_index("c")
    base = wid * _ROWS_PER_W
    pltpu.sync_copy(rw_hbm.at[pl.ds(base, _ROWS_PER_W)], rw_v)

    lane = lax.iota(jnp.int32, _L)
    lo8 = lane < 8
    neg_inf = jnp.float32(-jnp.inf)

    def sort_desc(x):
        return plsc.sort_key_val(x, x, descending=True)[0]

    def merge_desc(a, b):
        c = jnp.where(lo8, a, lax.rev(b, (0,)))
        return sort_desc(c)

    def row(r, _):
        v0 = rw_v[r, pl.ds(0, _L)]
        v1 = rw_v[r, pl.ds(16, _L)]
        v2 = rw_v[r, pl.ds(32, _L)]
        v3 = rw_v[r, pl.ds(48, _L)]
        m01 = merge_desc(sort_desc(v0), sort_desc(v1))
        m23 = merge_desc(sort_desc(v2), sort_desc(v3))
        m = merge_desc(m01, m23)
        rowmax = jnp.max(m)
        t8 = jnp.min(jnp.where(lo8, m, jnp.float32(jnp.inf)))
        e0 = jnp.where(v0 >= t8, jnp.exp(v0 - rowmax), 0.0)
        e1 = jnp.where(v1 >= t8, jnp.exp(v1 - rowmax), 0.0)
        e2 = jnp.where(v2 >= t8, jnp.exp(v2 - rowmax), 0.0)
        e3 = jnp.where(v3 >= t8, jnp.exp(v3 - rowmax), 0.0)
        den = jnp.broadcast_to(jnp.sum(e0 + e1 + e2 + e3), (_L,))
        scale = jnp.ones((_L,), jnp.float32) / den
        out_v[r, pl.ds(0, _L)] = e0 * scale
        out_v[r, pl.ds(16, _L)] = e1 * scale
        out_v[r, pl.ds(32, _L)] = e2 * scale
        out_v[r, pl.ds(48, _L)] = e3 * scale
        return _

    lax.fori_loop(0, _ROWS_PER_W, row, None)
    pltpu.sync_copy(out_v, out_hbm.at[pl.ds(base, _ROWS_PER_W)])


_route = pl.kernel(
    _route_body,
    out_type=jax.ShapeDtypeStruct((N_TOKENS, NUM_EXPERTS), jnp.float32),
    mesh=plsc.VectorSubcoreMesh(core_axis_name="c", subcore_axis_name="s"),
    scratch_types=[
        pltpu.VMEM((_ROWS_PER_W, NUM_EXPERTS), jnp.float32),
        pltpu.VMEM((_ROWS_PER_W, NUM_EXPERTS), jnp.float32),
    ],
    compiler_params=pltpu.CompilerParams(needs_layout_passes=False),
)


@jax.jit
def kernel(inputs, w_gate):
    rw = _matmul(inputs, w_gate)
    return _route(rw)


# speedup vs baseline: 4.4531x; 1.1470x over previous
import functools

import jax
import jax.numpy as jnp
from jax import lax
from jax.experimental import pallas as pl
from jax.experimental.pallas import tpu as pltpu
from jax.experimental.pallas import tpu_sc as plsc

N_TOKENS = 16384
D_MODEL = 4096
NUM_EXPERTS = 64
TOP_K = 8

_NC = 2
_NS = 16
_NW = _NC * _NS
_L = 16

_ROWS_PER_W = N_TOKENS // _NW


def _mm_body(x_ref, w_ref, o_ref):
    o_ref[...] = jnp.dot(x_ref[...], w_ref[...],
                         preferred_element_type=jnp.float32)


def _matmul(inputs, w_gate, bm=512):
    n, d = inputs.shape
    e = w_gate.shape[1]
    return pl.pallas_call(
        _mm_body,
        grid=(n // bm,),
        in_specs=[
            pl.BlockSpec((bm, d), lambda i: (i, 0)),
            pl.BlockSpec((d, e), lambda i: (0, 0)),
        ],
        out_specs=pl.BlockSpec((bm, e), lambda i: (i, 0)),
        out_shape=jax.ShapeDtypeStruct((n, e), jnp.float32),
    )(inputs, w_gate)


def _route_body(rw_hbm, out_hbm, rw_v, out_v):
    wid = lax.axis_index("s") * _NC + lax.axis_index("c")
    base = wid * _ROWS_PER_W
    pltpu.sync_copy(rw_hbm.at[pl.ds(base, _ROWS_PER_W)], rw_v)

    lane = lax.iota(jnp.int32, _L)
    lo8 = lane < 8
    neg_inf = jnp.float32(-jnp.inf)

    def sort_desc(x):
        return plsc.sort_key_val(x, x, descending=True)[0]

    def merge_desc(a, b):
        c = jnp.where(lo8, a, lax.rev(b, (0,)))
        return sort_desc(c)

    def row(r, _):
        v0 = rw_v[r, pl.ds(0, _L)]
        v1 = rw_v[r, pl.ds(16, _L)]
        v2 = rw_v[r, pl.ds(32, _L)]
        v3 = rw_v[r, pl.ds(48, _L)]
        m01 = merge_desc(sort_desc(v0), sort_desc(v1))
        m23 = merge_desc(sort_desc(v2), sort_desc(v3))
        m = merge_desc(m01, m23)
        rowmax = jnp.max(m)
        t8 = jnp.min(jnp.where(lo8, m, jnp.float32(jnp.inf)))
        e0 = jnp.where(v0 >= t8, jnp.exp(v0 - rowmax), 0.0)
        e1 = jnp.where(v1 >= t8, jnp.exp(v1 - rowmax), 0.0)
        e2 = jnp.where(v2 >= t8, jnp.exp(v2 - rowmax), 0.0)
        e3 = jnp.where(v3 >= t8, jnp.exp(v3 - rowmax), 0.0)
        den = jnp.broadcast_to(jnp.sum(e0 + e1 + e2 + e3), (_L,))
        scale = jnp.ones((_L,), jnp.float32) / den
        out_v[r, pl.ds(0, _L)] = e0 * scale
        out_v[r, pl.ds(16, _L)] = e1 * scale
        out_v[r, pl.ds(32, _L)] = e2 * scale
        out_v[r, pl.ds(48, _L)] = e3 * scale
        return _

    lax.fori_loop(0, _ROWS_PER_W, row, None)
    pltpu.sync_copy(out_v, out_hbm.at[pl.ds(base, _ROWS_PER_W)])


_route = pl.kernel(
    _route_body,
    out_type=jax.ShapeDtypeStruct((N_TOKENS, NUM_EXPERTS), jnp.float32),
    mesh=plsc.VectorSubcoreMesh(core_axis_name="c", subcore_axis_name="s"),
    scratch_types=[
        pltpu.VMEM((_ROWS_PER_W, NUM_EXPERTS), jnp.float32),
        pltpu.VMEM((_ROWS_PER_W, NUM_EXPERTS), jnp.float32),
    ],
    compiler_params=pltpu.CompilerParams(needs_layout_passes=False),
)


@jax.jit
def kernel(inputs, w_gate):
    rw = _matmul(inputs, w_gate)
    return _route(rw)
